# baseline (speedup 1.0000x reference)
import functools

import jax
import jax.numpy as jnp
from jax import lax
from jax.experimental import pallas as pl
from jax.experimental.pallas import tpu as pltpu
from jax.experimental.pallas import tpu_sc as plsc

_B = 16
_NPG = 1024
_K = 16
_NCLS = 40
_HID = 128
_NTOT = _B * _NPG
_E = _NTOT * _K
_TW = 16

_ACOS_C = (-0.0012624911, 0.0066700901, -0.0170881256, 0.0308918810,
           -0.0501743046, 0.0889789874, -0.2145988016, 1.5707963050)
_PI = 3.141592653589793


def _acos(v):
    t = jnp.minimum(jnp.abs(v), 1.0)
    p = jnp.float32(_ACOS_C[0])
    for c in _ACOS_C[1:]:
        p = p * t + jnp.float32(c)
    r = jnp.sqrt(jnp.maximum(1.0 - t, 0.0)) * p
    return jnp.where(v < 0.0, jnp.float32(_PI) - r, r)


def _sigmoid(v):
    return 1.0 / (1.0 + jnp.exp(-v))


def _softplus(v):
    return jnp.maximum(v, 0.0) + jnp.log1p(jnp.exp(-jnp.abs(v)))


def _sel_body(pos3_ref, posT_ref, out_ref):
    N = _NPG
    K = _K
    pos = pos3_ref[0]
    posT = posT_ref[0]
    d0 = pos[:, 0:1] - posT[0:1, :]
    d1 = pos[:, 1:2] - posT[1:2, :]
    d2 = pos[:, 2:3] - posT[2:3, :]
    D = (d0 * d0 + d1 * d1) + d2 * d2

    iota0 = lax.broadcasted_iota(jnp.int32, (N, N), 0)
    kio = lax.broadcasted_iota(jnp.int32, (K, N), 0)
    SCALE = jnp.float32(1 << 19)
    MAXI = jnp.int32(2147483647)
    keys = jnp.bitwise_or(
        lax.shift_left((D * SCALE).astype(jnp.int32), 10), iota0)

    def sel(k, carry):
        keys, idxT = carry
        mk = jnp.min(keys, axis=0, keepdims=True)
        keys = jnp.where(keys == mk, MAXI, keys)
        idxT = jnp.where(kio == k, jnp.bitwise_and(mk, 1023), idxT)
        return (keys, idxT)

    _, idxT = lax.fori_loop(0, K, sel, (keys, jnp.zeros((K, N), jnp.int32)))
    b = pl.program_id(0)
    out_ref[0] = idxT + b * N


def _select(pos3, posT):
    return pl.pallas_call(
        _sel_body,
        grid=(_B,),
        in_specs=[pl.BlockSpec((1, _NPG, 3), lambda b: (b, 0, 0)),
                  pl.BlockSpec((1, 3, _NPG), lambda b: (b, 0, 0))],
        out_specs=pl.BlockSpec((1, _K, _NPG), lambda b: (b, 0, 0)),
        out_shape=jax.ShapeDtypeStruct((_B, _K, _NPG), jnp.int32),
    )(pos3, posT)


def _sc_gather(table, idx_flat, C):
    info = plsc.get_sparse_core_info()
    NW = info.num_cores * info.num_subcores
    L = info.num_lanes
    b_per_w = _E // NW
    CHUNK = 2048
    NCH = b_per_w // CHUNK
    mesh = plsc.VectorSubcoreMesh(core_axis_name="c", subcore_axis_name="s")

    @functools.partial(
        pl.kernel, mesh=mesh,
        out_type=jax.ShapeDtypeStruct((C * _E,), jnp.float32),
        scratch_types=[
            pltpu.VMEM((CHUNK,), jnp.int32),
            pltpu.VMEM((C * CHUNK,), jnp.int32),
            pltpu.VMEM((C * CHUNK,), jnp.float32),
            pltpu.SemaphoreType.DMA,
        ],
    )
    def k(table_hbm, idx_hbm, out_hbm, idx_v, ivf_v, gout_v, sem):
        wid = lax.axis_index("s") * info.num_cores + lax.axis_index("c")
        base = wid * b_per_w
        for j in range(NCH):
            off = base + j * CHUNK
            pltpu.sync_copy(idx_hbm.at[pl.ds(off, CHUNK)], idx_v)

            def group(g, _):
                gi = idx_v[pl.ds(g * L, L)] * C
                for c in range(C):
                    ivf_v[pl.ds(c * CHUNK + g * L, L)] = gi + c
                return 0

            lax.fori_loop(0, CHUNK // L, group, 0)
            pltpu.async_copy(table_hbm.at[ivf_v], gout_v, sem).wait()
            for c in range(C):
                pltpu.sync_copy(gout_v.at[pl.ds(c * CHUNK, CHUNK)],
                                out_hbm.at[pl.ds(c * _E + off, CHUNK)])

    return k(table, idx_flat)


def _layer_compute(hT, gx, ea, wfT, bfT, wsT, bsT):
    f32 = jnp.float32
    hf = jnp.dot(wfT[:, 0:3], hT, preferred_element_type=f32) + bfT
    hs = jnp.dot(wsT[:, 0:3], hT, preferred_element_type=f32) + bsT
    aggs = []
    for c in range(3):
        F = hf[c:c + 1, :]
        S = hs[c:c + 1, :]
        for t in range(3):
            F = F + gx[t] * wfT[c:c + 1, 3 + t:4 + t]
            S = S + gx[t] * wsT[c:c + 1, 3 + t:4 + t]
        for t in range(4):
            F = F + ea[t] * wfT[c:c + 1, 6 + t:7 + t]
            S = S + ea[t] * wsT[c:c + 1, 6 + t:7 + t]
        m = _sigmoid(F) * _softplus(S)
        aggs.append(jnp.sum(m, axis=0, keepdims=True))
    return hT + jnp.concatenate(aggs, axis=0)


def _make_layer_body(mode):
    def body(*refs):
        f32 = jnp.float32
        it = iter(refs)
        gcm = next(it)[...][:, 0]
        if mode == 1:
            posT = next(it)[0]
        else:
            ea_in = next(it)[0]
        hT = next(it)[0]
        wfT = next(it)[...]
        bfT = next(it)[...]
        wsT = next(it)[...]
        bsT = next(it)[...]
        linT = next(it)[...]
        linb = next(it)[...]
        if mode == 3:
            clsT = next(it)[...]
            clsb = next(it)[...]
            p1 = next(it)[0]
            p2 = next(it)[0]
        outs = list(it)

        if mode == 1:
            gp = (gcm[0], gcm[1], gcm[2])
            gx = (gcm[3], gcm[4], gcm[5])
            v0 = gp[0] - posT[0:1, :]
            v1 = gp[1] - posT[1:2, :]
            v2 = gp[2] - posT[2:3, :]
            dist = jnp.sqrt((v0 * v0 + v1 * v1) + v2 * v2)
            ea = (_acos(v0), _acos(v1), _acos(v2), dist)
        else:
            gx = (gcm[0], gcm[1], gcm[2])
            ea = (ea_in[0], ea_in[1], ea_in[2], ea_in[3])

        hT_new = _layer_compute(hT, gx, ea, wfT, bfT, wsT, bsT)
        xl = jnp.dot(linT, hT_new, preferred_element_type=f32) + linb
        p = jnp.max(xl, axis=1, keepdims=True)

        if mode == 1:
            ea_ref, h_ref, p_ref = outs
            ea_ref[0] = jnp.concatenate(
                [e[None] for e in ea], axis=0)
            h_ref[0] = hT_new
            p_ref[0] = p
        elif mode == 2:
            h_ref, p_ref = outs
            h_ref[0] = hT_new
            p_ref[0] = p
        else:
            (o_ref,) = outs
            psum = (p1 + p2) + p
            o_ref[0] = (jnp.dot(clsT, psum, preferred_element_type=f32)
                        + clsb)
    return body


def _full_spec(arr):
    nd = arr.ndim
    return pl.BlockSpec(arr.shape, lambda b, _nd=nd: (0,) * _nd)


def _graph_spec(shape):
    nd = len(shape)
    return pl.BlockSpec((1,) + tuple(shape),
                        lambda b, _nd=nd: (b,) + (0,) * _nd)


def _run_layer(mode, gcm, aux, hT, wfT, bfT, wsT, bsT, linT, linb,
               cls_args=(), p_args=()):
    f32 = jnp.float32
    B, N, K = _B, _NPG, _K
    in_arrays = [gcm, aux, hT, wfT, bfT, wsT, bsT, linT, linb,
                 *cls_args, *p_args]
    C = gcm.shape[0]
    in_specs = ([pl.BlockSpec((C, 1, K, N), lambda b: (0, b, 0, 0)),
                 _graph_spec(aux.shape[1:]),
                 _graph_spec((3, N))]
                + [_full_spec(w) for w in (wfT, bfT, wsT, bsT, linT, linb)]
                + [_full_spec(w) for w in cls_args]
                + [_graph_spec(p.shape[1:]) for p in p_args])
    if mode == 1:
        out_shape = (jax.ShapeDtypeStruct((B, 4, K, N), f32),
                     jax.ShapeDtypeStruct((B, 3, N), f32),
                     jax.ShapeDtypeStruct((B, _HID, 1), f32))
        out_specs = (_graph_spec((4, K, N)), _graph_spec((3, N)),
                     _graph_spec((_HID, 1)))
    elif mode == 2:
        out_shape = (jax.ShapeDtypeStruct((B, 3, N), f32),
                     jax.ShapeDtypeStruct((B, _HID, 1), f32))
        out_specs = (_graph_spec((3, N)), _graph_spec((_HID, 1)))
    else:
        out_shape = jax.ShapeDtypeStruct((B, _NCLS, 1), f32)
        out_specs = _graph_spec((_NCLS, 1))
    return pl.pallas_call(
        _make_layer_body(mode),
        grid=(B,),
        in_specs=in_specs,
        out_specs=out_specs,
        out_shape=out_shape,
    )(*in_arrays)


def kernel(x, pos, batch, W_f1, b_f1, W_s1, b_s1, W_f2, b_f2, W_s2, b_s2,
           W_f3, b_f3, W_s3, b_s3, lin_W, lin_b, cls_W, cls_b):
    B, N = _B, _NPG
    f32 = jnp.float32

    pos3 = pos.reshape(B, N, 3)
    posT = jnp.transpose(pos3, (0, 2, 1))
    xT = jnp.transpose(x.reshape(B, N, 3), (0, 2, 1))

    idx_flat = _select(pos3, posT).reshape(_E)

    w = {}
    for i, (Wf, bf, Ws, bs) in enumerate(((W_f1, b_f1, W_s1, b_s1),
                                          (W_f2, b_f2, W_s2, b_s2),
                                          (W_f3, b_f3, W_s3, b_s3)), 1):
        w[i] = (Wf.T, bf.reshape(3, 1), Ws.T, bs.reshape(3, 1))
    linT, linb = lin_W.T, lin_b.reshape(_HID, 1)
    clsT, clsb = cls_W.T, cls_b.reshape(_NCLS, 1)

    g1 = _sc_gather(jnp.concatenate([pos, x], axis=1).reshape(-1),
                    idx_flat, 6).reshape(6, B, _K, N)
    ea, h1, p1 = _run_layer(1, g1, posT, xT, *w[1], linT, linb)

    g2 = _sc_gather(jnp.transpose(h1, (0, 2, 1)).reshape(-1),
                    idx_flat, 3).reshape(3, B, _K, N)
    h2, p2 = _run_layer(2, g2, ea, h1, *w[2], linT, linb)

    g3 = _sc_gather(jnp.transpose(h2, (0, 2, 1)).reshape(-1),
                    idx_flat, 3).reshape(3, B, _K, N)
    out = _run_layer(3, g3, ea, h2, *w[3], linT, linb,
                     cls_args=(clsT, clsb), p_args=(p1, p2))
    return out.reshape(B, _NCLS)


# speedup vs baseline: 1.1464x; 1.1464x over previous
import functools

import jax
import jax.numpy as jnp
from jax import lax
from jax.experimental import pallas as pl
from jax.experimental.pallas import tpu as pltpu
from jax.experimental.pallas import tpu_sc as plsc

_B = 16
_NPG = 1024
_K = 16
_NCLS = 40
_HID = 128
_NTOT = _B * _NPG
_E = _NTOT * _K
_TW = 16

_ACOS_C = (-0.0012624911, 0.0066700901, -0.0170881256, 0.0308918810,
           -0.0501743046, 0.0889789874, -0.2145988016, 1.5707963050)
_PI = 3.141592653589793


def _acos(v):
    t = jnp.minimum(jnp.abs(v), 1.0)
    p = jnp.float32(_ACOS_C[0])
    for c in _ACOS_C[1:]:
        p = p * t + jnp.float32(c)
    r = jnp.sqrt(jnp.maximum(1.0 - t, 0.0)) * p
    return jnp.where(v < 0.0, jnp.float32(_PI) - r, r)


def _sigmoid(v):
    return 1.0 / (1.0 + jnp.exp(-v))


def _softplus(v):
    return jnp.maximum(v, 0.0) + jnp.log1p(jnp.exp(-jnp.abs(v)))


def _sel1_body(pos3_ref, posT_ref, xT_ref, wf_ref, bf_ref, ws_ref, bs_ref,
               linT_ref, linb_ref, idx_ref, ea_ref, h_ref, p_ref):
    N = _NPG
    K = _K
    f32 = jnp.float32
    pos = pos3_ref[0]
    posT = posT_ref[0]
    xT = xT_ref[0]
    d0 = pos[:, 0:1] - posT[0:1, :]
    d1 = pos[:, 1:2] - posT[1:2, :]
    d2 = pos[:, 2:3] - posT[2:3, :]
    D = (d0 * d0 + d1 * d1) + d2 * d2

    iota0 = lax.broadcasted_iota(jnp.int32, (N, N), 0)
    kio = lax.broadcasted_iota(jnp.int32, (K, N), 0)
    SCALE = jnp.float32(1 << 19)
    MAXI = jnp.int32(2147483647)
    keys = jnp.bitwise_or(
        lax.shift_left((D * SCALE).astype(jnp.int32), 10), iota0)

    def sel(k, carry):
        keys, idxT = carry
        mk = jnp.min(keys, axis=0, keepdims=True)
        keys = jnp.where(keys == mk, MAXI, keys)
        idxT = jnp.where(kio == k, jnp.bitwise_and(mk, 1023), idxT)
        return (keys, idxT)

    _, idxT = lax.fori_loop(0, K, sel, (keys, jnp.zeros((K, N), jnp.int32)))
    b = pl.program_id(0)
    idx_ref[0] = idxT + b * N

    idx_q = lax.shift_right_logical(idxT, 7)
    idx_r = jnp.bitwise_and(idxT, 127)
    z_kn = jnp.zeros((K, N), f32)

    def grow(rowT):
        acc = z_kn
        for c in range(N // 128):
            xc = jnp.broadcast_to(rowT[:, c * 128:(c + 1) * 128], (K, 128))
            gc = jnp.take_along_axis(xc, idx_r, axis=1)
            acc = jnp.where(idx_q == c, gc, acc)
        return acc

    v0 = grow(posT[0:1, :]) - posT[0:1, :]
    v1 = grow(posT[1:2, :]) - posT[1:2, :]
    v2 = grow(posT[2:3, :]) - posT[2:3, :]
    dist = jnp.sqrt((v0 * v0 + v1 * v1) + v2 * v2)
    ea = (_acos(v0), _acos(v1), _acos(v2), dist)
    gx = (grow(xT[0:1, :]), grow(xT[1:2, :]), grow(xT[2:3, :]))

    hT_new = _layer_compute(xT, gx, ea, wf_ref[...], bf_ref[...],
                            ws_ref[...], bs_ref[...])
    xl = (jnp.dot(linT_ref[...], hT_new, preferred_element_type=f32)
          + linb_ref[...])
    ea_ref[0] = jnp.concatenate([e[None] for e in ea], axis=0)
    h_ref[0] = hT_new
    p_ref[0] = jnp.max(xl, axis=1, keepdims=True)


def _select_layer1(pos3, posT, xT, wfT, bfT, wsT, bsT, linT, linb):
    B, N, K = _B, _NPG, _K
    f32 = jnp.float32
    return pl.pallas_call(
        _sel1_body,
        grid=(B,),
        in_specs=([pl.BlockSpec((1, N, 3), lambda b: (b, 0, 0)),
                   pl.BlockSpec((1, 3, N), lambda b: (b, 0, 0)),
                   pl.BlockSpec((1, 3, N), lambda b: (b, 0, 0))]
                  + [_full_spec(w) for w in
                     (wfT, bfT, wsT, bsT, linT, linb)]),
        out_specs=(_graph_spec((K, N)), _graph_spec((4, K, N)),
                   _graph_spec((3, N)), _graph_spec((_HID, 1))),
        out_shape=(jax.ShapeDtypeStruct((B, K, N), jnp.int32),
                   jax.ShapeDtypeStruct((B, 4, K, N), f32),
                   jax.ShapeDtypeStruct((B, 3, N), f32),
                   jax.ShapeDtypeStruct((B, _HID, 1), f32)),
    )(pos3, posT, xT, wfT, bfT, wsT, bsT, linT, linb)


def _sc_gather(table, idx_flat, C):
    info = plsc.get_sparse_core_info()
    NW = info.num_cores * info.num_subcores
    L = info.num_lanes
    b_per_w = _E // NW
    CHUNK = 2048
    NCH = b_per_w // CHUNK
    mesh = plsc.VectorSubcoreMesh(core_axis_name="c", subcore_axis_name="s")

    @functools.partial(
        pl.kernel, mesh=mesh,
        out_type=jax.ShapeDtypeStruct((C * _E,), jnp.float32),
        scratch_types=[
            pltpu.VMEM((CHUNK,), jnp.int32),
            pltpu.VMEM((C * CHUNK,), jnp.int32),
            pltpu.VMEM((C * CHUNK,), jnp.float32),
            pltpu.SemaphoreType.DMA,
        ],
    )
    def k(table_hbm, idx_hbm, out_hbm, idx_v, ivf_v, gout_v, sem):
        wid = lax.axis_index("s") * info.num_cores + lax.axis_index("c")
        base = wid * b_per_w
        for j in range(NCH):
            off = base + j * CHUNK
            pltpu.sync_copy(idx_hbm.at[pl.ds(off, CHUNK)], idx_v)

            def group(g, _):
                gi = idx_v[pl.ds(g * L, L)] * C
                for c in range(C):
                    ivf_v[pl.ds(c * CHUNK + g * L, L)] = gi + c
                return 0

            lax.fori_loop(0, CHUNK // L, group, 0)
            pltpu.async_copy(table_hbm.at[ivf_v], gout_v, sem).wait()
            for c in range(C):
                pltpu.sync_copy(gout_v.at[pl.ds(c * CHUNK, CHUNK)],
                                out_hbm.at[pl.ds(c * _E + off, CHUNK)])

    return k(table, idx_flat)


def _layer_compute(hT, gx, ea, wfT, bfT, wsT, bsT):
    f32 = jnp.float32
    hf = jnp.dot(wfT[:, 0:3], hT, preferred_element_type=f32) + bfT
    hs = jnp.dot(wsT[:, 0:3], hT, preferred_element_type=f32) + bsT
    aggs = []
    for c in range(3):
        F = hf[c:c + 1, :]
        S = hs[c:c + 1, :]
        for t in range(3):
            F = F + gx[t] * wfT[c:c + 1, 3 + t:4 + t]
            S = S + gx[t] * wsT[c:c + 1, 3 + t:4 + t]
        for t in range(4):
            F = F + ea[t] * wfT[c:c + 1, 6 + t:7 + t]
            S = S + ea[t] * wsT[c:c + 1, 6 + t:7 + t]
        m = _sigmoid(F) * _softplus(S)
        aggs.append(jnp.sum(m, axis=0, keepdims=True))
    return hT + jnp.concatenate(aggs, axis=0)


def _make_layer_body(mode):
    def body(*refs):
        f32 = jnp.float32
        it = iter(refs)
        gcm = next(it)[...][:, 0]
        if mode == 1:
            posT = next(it)[0]
        else:
            ea_in = next(it)[0]
        hT = next(it)[0]
        wfT = next(it)[...]
        bfT = next(it)[...]
        wsT = next(it)[...]
        bsT = next(it)[...]
        linT = next(it)[...]
        linb = next(it)[...]
        if mode == 3:
            clsT = next(it)[...]
            clsb = next(it)[...]
            p1 = next(it)[0]
            p2 = next(it)[0]
        outs = list(it)

        if mode == 1:
            gp = (gcm[0], gcm[1], gcm[2])
            gx = (gcm[3], gcm[4], gcm[5])
            v0 = gp[0] - posT[0:1, :]
            v1 = gp[1] - posT[1:2, :]
            v2 = gp[2] - posT[2:3, :]
            dist = jnp.sqrt((v0 * v0 + v1 * v1) + v2 * v2)
            ea = (_acos(v0), _acos(v1), _acos(v2), dist)
        else:
            gx = (gcm[0], gcm[1], gcm[2])
            ea = (ea_in[0], ea_in[1], ea_in[2], ea_in[3])

        hT_new = _layer_compute(hT, gx, ea, wfT, bfT, wsT, bsT)
        xl = jnp.dot(linT, hT_new, preferred_element_type=f32) + linb
        p = jnp.max(xl, axis=1, keepdims=True)

        if mode == 1:
            ea_ref, h_ref, p_ref = outs
            ea_ref[0] = jnp.concatenate(
                [e[None] for e in ea], axis=0)
            h_ref[0] = hT_new
            p_ref[0] = p
        elif mode == 2:
            h_ref, p_ref = outs
            h_ref[0] = hT_new
            p_ref[0] = p
        else:
            (o_ref,) = outs
            psum = (p1 + p2) + p
            o_ref[0] = (jnp.dot(clsT, psum, preferred_element_type=f32)
                        + clsb)
    return body


def _full_spec(arr):
    nd = arr.ndim
    return pl.BlockSpec(arr.shape, lambda b, _nd=nd: (0,) * _nd)


def _graph_spec(shape):
    nd = len(shape)
    return pl.BlockSpec((1,) + tuple(shape),
                        lambda b, _nd=nd: (b,) + (0,) * _nd)


def _run_layer(mode, gcm, aux, hT, wfT, bfT, wsT, bsT, linT, linb,
               cls_args=(), p_args=()):
    f32 = jnp.float32
    B, N, K = _B, _NPG, _K
    in_arrays = [gcm, aux, hT, wfT, bfT, wsT, bsT, linT, linb,
                 *cls_args, *p_args]
    C = gcm.shape[0]
    in_specs = ([pl.BlockSpec((C, 1, K, N), lambda b: (0, b, 0, 0)),
                 _graph_spec(aux.shape[1:]),
                 _graph_spec((3, N))]
                + [_full_spec(w) for w in (wfT, bfT, wsT, bsT, linT, linb)]
                + [_full_spec(w) for w in cls_args]
                + [_graph_spec(p.shape[1:]) for p in p_args])
    if mode == 1:
        out_shape = (jax.ShapeDtypeStruct((B, 4, K, N), f32),
                     jax.ShapeDtypeStruct((B, 3, N), f32),
                     jax.ShapeDtypeStruct((B, _HID, 1), f32))
        out_specs = (_graph_spec((4, K, N)), _graph_spec((3, N)),
                     _graph_spec((_HID, 1)))
    elif mode == 2:
        out_shape = (jax.ShapeDtypeStruct((B, 3, N), f32),
                     jax.ShapeDtypeStruct((B, _HID, 1), f32))
        out_specs = (_graph_spec((3, N)), _graph_spec((_HID, 1)))
    else:
        out_shape = jax.ShapeDtypeStruct((B, _NCLS, 1), f32)
        out_specs = _graph_spec((_NCLS, 1))
    return pl.pallas_call(
        _make_layer_body(mode),
        grid=(B,),
        in_specs=in_specs,
        out_specs=out_specs,
        out_shape=out_shape,
    )(*in_arrays)


def kernel(x, pos, batch, W_f1, b_f1, W_s1, b_s1, W_f2, b_f2, W_s2, b_s2,
           W_f3, b_f3, W_s3, b_s3, lin_W, lin_b, cls_W, cls_b):
    B, N = _B, _NPG
    f32 = jnp.float32

    pos3 = pos.reshape(B, N, 3)
    posT = jnp.transpose(pos3, (0, 2, 1))
    xT = jnp.transpose(x.reshape(B, N, 3), (0, 2, 1))


    w = {}
    for i, (Wf, bf, Ws, bs) in enumerate(((W_f1, b_f1, W_s1, b_s1),
                                          (W_f2, b_f2, W_s2, b_s2),
                                          (W_f3, b_f3, W_s3, b_s3)), 1):
        w[i] = (Wf.T, bf.reshape(3, 1), Ws.T, bs.reshape(3, 1))
    linT, linb = lin_W.T, lin_b.reshape(_HID, 1)
    clsT, clsb = cls_W.T, cls_b.reshape(_NCLS, 1)

    idx, ea, h1, p1 = _select_layer1(pos3, posT, xT, *w[1], linT, linb)
    idx_flat = idx.reshape(_E)

    g2 = _sc_gather(jnp.transpose(h1, (0, 2, 1)).reshape(-1),
                    idx_flat, 3).reshape(3, B, _K, N)
    h2, p2 = _run_layer(2, g2, ea, h1, *w[2], linT, linb)

    g3 = _sc_gather(jnp.transpose(h2, (0, 2, 1)).reshape(-1),
                    idx_flat, 3).reshape(3, B, _K, N)
    out = _run_layer(3, g3, ea, h2, *w[3], linT, linb,
                     cls_args=(clsT, clsb), p_args=(p1, p2))
    return out.reshape(B, _NCLS)
